# baseline (speedup 1.0000x reference)
import functools

import jax
import jax.numpy as jnp
from jax import lax
from jax.experimental import pallas as pl
from jax.experimental.pallas import tpu as pltpu
from jax.experimental.pallas import tpu_sc as plsc

N = 100000
E = 6400000
NP = 100352
EB = 25600
NW = 32
EPW = E // NW
KG = 4000
UG = 5
KS = 6400
U = 10

_MESH = plsc.VectorSubcoreMesh(core_axis_name="c", subcore_axis_name="s")
_SC_PARAMS = pltpu.CompilerParams(needs_layout_passes=False)


def _wid():
    return lax.axis_index("s") * 2 + lax.axis_index("c")


def _gather_body(table_hbm, idx_hbm, out_hbm, tab_v, idx_v0, idx_v1,
                 out_v0, out_v1, isem0, isem1, osem0, osem1):
    w = _wid()
    base = w * EPW
    pltpu.sync_copy(table_hbm, tab_v)
    nchunks = EPW // KG
    isems = (isem0, isem1)
    osems = (osem0, osem1)
    idxs = (idx_v0, idx_v1)
    outs = (out_v0, out_v1)

    def start_in(j, b):
        pltpu.make_async_copy(
            idx_hbm.at[pl.ds(base + j * KG, KG)], idxs[b], isems[b]).start()

    def wait_in(b):
        pltpu.make_async_copy(
            idx_hbm.at[pl.ds(0, KG)], idxs[b], isems[b]).wait()

    def start_out(j, b):
        pltpu.make_async_copy(
            outs[b], out_hbm.at[pl.ds(base + j * KG, KG)], osems[b]).start()

    def wait_out(b):
        pltpu.make_async_copy(
            outs[b], out_hbm.at[pl.ds(0, KG)], osems[b]).wait()

    def process(b):
        def group(g, carry):
            bs = g * (16 * UG)
            ivs = [idxs[b][pl.ds(bs + u * 16, 16)] for u in range(UG)]
            vals = [plsc.load_gather(tab_v, [ivs[u]]) for u in range(UG)]
            for u in range(UG):
                outs[b][pl.ds(bs + u * 16, 16)] = vals[u]
            return carry

        lax.fori_loop(0, KG // (16 * UG), group, 0)

    start_in(0, 0)
    start_in(1, 1)

    def pair(jj, carry):
        for b in (0, 1):
            j = jj * 2 + b
            wait_in(b)

            @pl.when(j >= 2)
            def _():
                wait_out(b)

            process(b)
            start_out(j, b)

            @pl.when(j + 2 < nchunks)
            def _():
                start_in(j + 2, b)
        return carry

    lax.fori_loop(0, nchunks // 2, pair, 0)
    wait_out(0)
    wait_out(1)


_sc_gather = pl.kernel(
    _gather_body,
    mesh=_MESH,
    out_type=jax.ShapeDtypeStruct((E,), jnp.float32),
    scratch_types=[
        pltpu.VMEM((NP,), jnp.float32),
        pltpu.VMEM((KG,), jnp.int32),
        pltpu.VMEM((KG,), jnp.int32),
        pltpu.VMEM((KG,), jnp.float32),
        pltpu.VMEM((KG,), jnp.float32),
        pltpu.SemaphoreType.DMA,
        pltpu.SemaphoreType.DMA,
        pltpu.SemaphoreType.DMA,
        pltpu.SemaphoreType.DMA,
    ],
    compiler_params=_SC_PARAMS,
)


def _scatter_body(dst_hbm, *rest):
    msg_refs = rest[:32]
    out_hbm, aggr_v, idx_v, val_v, sem0, sem1 = rest[32:]
    ch = _wid()

    zeros = jnp.zeros((16,), jnp.float32)

    def zinit(i, carry):
        aggr_v[pl.ds(i * 16, 16)] = zeros
        return carry

    lax.fori_loop(0, NP // 16, zinit, 0)

    nchunks = E // KS
    sems = (sem0, sem1)

    def start(j, b):
        pltpu.make_async_copy(
            dst_hbm.at[pl.ds(j * KS, KS)], idx_v.at[b], sems[b]).start()
        for cc in range(32):
            @pl.when(ch == cc)
            def _(_cc=cc):
                pltpu.make_async_copy(
                    msg_refs[_cc].at[pl.ds(j * KS, KS)], val_v.at[b],
                    sems[b]).start()

    def wait(b):
        pltpu.make_async_copy(
            dst_hbm.at[pl.ds(0, KS)], idx_v.at[b], sems[b]).wait()
        pltpu.make_async_copy(
            msg_refs[0].at[pl.ds(0, KS)], val_v.at[b], sems[b]).wait()

    def process(b):
        def group(g, acc):
            base = g * (16 * U)
            ivs = [idx_v[b, pl.ds(base + u * 16, 16)] for u in range(U)]
            mvs = [val_v[b, pl.ds(base + u * 16, 16)] for u in range(U)]
            curs = [plsc.load_gather(aggr_v, [ivs[u]]) for u in range(U)]
            news = [jnp.maximum(curs[u], mvs[u]) for u in range(U)]
            pends = [mvs[u] > curs[u] for u in range(U)]
            for u in range(U):
                plsc.store_scatter(aggr_v, [ivs[u]], news[u], mask=pends[u])
            chks = [plsc.load_gather(aggr_v, [ivs[u]]) for u in range(U)]
            for u in range(U):
                acc = jnp.logical_or(
                    acc, jnp.logical_and(pends[u], chks[u] < news[u]))
            return acc

        acc = lax.fori_loop(0, KS // (16 * U), group,
                            jnp.zeros((16,), jnp.bool_))

        def walk_cond(a):
            return jnp.any(a)

        def walk(a):
            def wgroup(g, acc2):
                base = g * (16 * U)
                for u in range(U):
                    iv = idx_v[b, pl.ds(base + u * 16, 16)]
                    mv = val_v[b, pl.ds(base + u * 16, 16)]
                    cell = plsc.load_gather(aggr_v, [iv])
                    pend = mv > cell
                    plsc.store_scatter(aggr_v, [iv], mv, mask=pend)
                    chk = plsc.load_gather(aggr_v, [iv])
                    acc2 = jnp.logical_or(
                        acc2, jnp.logical_and(pend, chk < mv))
                return acc2

            return lax.fori_loop(0, KS // (16 * U), wgroup,
                                 jnp.zeros((16,), jnp.bool_))

        lax.while_loop(walk_cond, walk, acc)

    start(0, 0)
    start(1, 1)

    def pair(jj, carry):
        for b in (0, 1):
            j = jj * 2 + b
            wait(b)
            process(b)

            @pl.when(j + 2 < nchunks)
            def _():
                start(j + 2, b)
        return carry

    lax.fori_loop(0, nchunks // 2, pair, 0)
    pltpu.sync_copy(aggr_v, out_hbm.at[pl.ds(ch * NP, NP)])


_sc_scatter_max = pl.kernel(
    _scatter_body,
    mesh=_MESH,
    out_type=jax.ShapeDtypeStruct((32 * NP,), jnp.float32),
    scratch_types=[
        pltpu.VMEM((NP,), jnp.float32),
        pltpu.VMEM((2, KS), jnp.int32),
        pltpu.VMEM((2, KS), jnp.float32),
        pltpu.SemaphoreType.DMA,
        pltpu.SemaphoreType.DMA,
    ],
    compiler_params=_SC_PARAMS,
)


def _edge_mlp_body(g0, g1, t, ea, w1a_r, b1a_r, w1b_r, b1b_r, *outs):
    x3 = jnp.concatenate(
        [g0[...].reshape(1, EB), g1[...].reshape(1, EB), t[...].reshape(1, EB)],
        axis=0,
    )
    w1a = w1a_r[...]
    h = jax.lax.dot_general(w1a[:3, :], x3, (((0,), (0,)), ((), ())),
                            preferred_element_type=jnp.float32)
    h = h + jax.lax.dot_general(w1a[3:, :], ea[...], (((0,), (0,)), ((), ())),
                                preferred_element_type=jnp.float32)
    h = jnp.maximum(h + b1a_r[...].reshape(16, 1), 0.0)
    m = jnp.maximum(
        jax.lax.dot_general(w1b_r[...], h, (((0,), (0,)), ((), ())),
                            preferred_element_type=jnp.float32)
        + b1b_r[...].reshape(32, 1), 0.0)
    for c in range(32):
        outs[c][...] = m[c, :]


def _edge_mlp(g0, g1, t, ea_t, w1a, b1a, w1b, b1b):
    grid = (E // EB,)
    vec = pl.BlockSpec((EB,), lambda i: (i,))
    return pl.pallas_call(
        _edge_mlp_body,
        grid=grid,
        in_specs=[
            vec, vec, vec,
            pl.BlockSpec((2, EB), lambda i: (0, i)),
            pl.BlockSpec((5, 16), lambda i: (0, 0)),
            pl.BlockSpec((16,), lambda i: (0,)),
            pl.BlockSpec((16, 32), lambda i: (0, 0)),
            pl.BlockSpec((32,), lambda i: (0,)),
        ],
        out_specs=[pl.BlockSpec((EB,), lambda i: (i,)) for _ in range(32)],
        out_shape=[jax.ShapeDtypeStruct((E,), jnp.float32) for _ in range(32)],
    )(g0, g1, t, ea_t, w1a, b1a, w1b, b1b)


def _node_mlp_body(x_ref, aggr_ref, w2a_ref, b2a_ref, w2b_ref, b2b_ref, out_ref):
    xa = x_ref[...]
    ag = aggr_ref[...]
    wa = w2a_ref[...]
    h = jax.lax.dot_general(wa[:3, :], xa, (((0,), (0,)), ((), ())),
                            preferred_element_type=jnp.float32)
    h = h + jax.lax.dot_general(wa[3:, :], ag, (((0,), (0,)), ((), ())),
                                preferred_element_type=jnp.float32)
    h = jnp.maximum(h + b2a_ref[...].reshape(16, 1), 0.0)
    o = jax.lax.dot_general(w2b_ref[...], h, (((0,), (0,)), ((), ())),
                            preferred_element_type=jnp.float32)
    out_ref[...] = jax.nn.sigmoid(o + b2b_ref[...].reshape(1, 1))


def _node_mlp(x_t, aggr_t, w2a, b2a, w2b, b2b):
    NB = 12544
    grid = (NP // NB,)
    return pl.pallas_call(
        _node_mlp_body,
        grid=grid,
        in_specs=[
            pl.BlockSpec((3, NB), lambda i: (0, i)),
            pl.BlockSpec((32, NB), lambda i: (0, i)),
            pl.BlockSpec((35, 16), lambda i: (0, 0)),
            pl.BlockSpec((16,), lambda i: (0,)),
            pl.BlockSpec((16, 1), lambda i: (0, 0)),
            pl.BlockSpec((1,), lambda i: (0,)),
        ],
        out_specs=pl.BlockSpec((1, NB), lambda i: (0, i)),
        out_shape=jax.ShapeDtypeStruct((1, NP), jnp.float32),
    )(x_t, aggr_t, w2a, b2a, w2b, b2b)


def kernel(x, edge_index, edge_attr, w1a, b1a, w1b, b1b, w2a, b2a, w2b, b2b):
    src = edge_index[0].astype(jnp.int32)
    dst = edge_index[1].astype(jnp.int32)
    ea_t = edge_attr.T

    pad = (0, NP - N)
    g0 = _sc_gather(jnp.pad(x[:, 0], pad), src)
    g1 = _sc_gather(jnp.pad(x[:, 1], pad), src)
    col2_tab = jnp.pad(x[:, 2], pad)

    x_t = jnp.pad(x.T, ((0, 0), pad))

    for _ in range(3):
        t_e = _sc_gather(col2_tab, src)
        msgs = _edge_mlp(g0, g1, t_e, ea_t, w1a, b1a, w1b, b1b)
        aggr_flat = _sc_scatter_max(dst, *msgs)
        aggr_t = aggr_flat.reshape(32, NP)
        comb = _node_mlp(x_t, aggr_t, w2a, b2a, w2b, b2b)
        col2_tab = comb.reshape(NP)
        x_t = x_t.at[2, :].set(comb[0])

    out = jnp.concatenate([x[:, :2], col2_tab[:N, None]], axis=1)
    return out


# speedup vs baseline: 1.2355x; 1.2355x over previous
import functools

import jax
import jax.numpy as jnp
from jax import lax
from jax.experimental import pallas as pl
from jax.experimental.pallas import tpu as pltpu
from jax.experimental.pallas import tpu_sc as plsc

N = 100000
E = 6400000
NP = 100352
EB = 25600
NW = 32
EPW = E // NW
KG = 4000
UG = 5
KS = 6400
U = 8

_MESH = plsc.VectorSubcoreMesh(core_axis_name="c", subcore_axis_name="s")
_SC_PARAMS = pltpu.CompilerParams(needs_layout_passes=False)


def _wid():
    return lax.axis_index("s") * 2 + lax.axis_index("c")


def _gather_body(table_hbm, idx_hbm, out_hbm, tab_v, idx_v0, idx_v1,
                 out_v0, out_v1, isem0, isem1, osem0, osem1):
    w = _wid()
    base = w * EPW
    pltpu.sync_copy(table_hbm, tab_v)
    nchunks = EPW // KG
    isems = (isem0, isem1)
    osems = (osem0, osem1)
    idxs = (idx_v0, idx_v1)
    outs = (out_v0, out_v1)

    def start_in(j, b):
        pltpu.make_async_copy(
            idx_hbm.at[pl.ds(base + j * KG, KG)], idxs[b], isems[b]).start()

    def wait_in(b):
        pltpu.make_async_copy(
            idx_hbm.at[pl.ds(0, KG)], idxs[b], isems[b]).wait()

    def start_out(j, b):
        pltpu.make_async_copy(
            outs[b], out_hbm.at[pl.ds(base + j * KG, KG)], osems[b]).start()

    def wait_out(b):
        pltpu.make_async_copy(
            outs[b], out_hbm.at[pl.ds(0, KG)], osems[b]).wait()

    def process(b):
        def group(g, carry):
            bs = g * (16 * UG)
            ivs = [idxs[b][pl.ds(bs + u * 16, 16)] for u in range(UG)]
            vals = [plsc.load_gather(tab_v, [ivs[u]]) for u in range(UG)]
            for u in range(UG):
                outs[b][pl.ds(bs + u * 16, 16)] = vals[u]
            return carry

        lax.fori_loop(0, KG // (16 * UG), group, 0)

    start_in(0, 0)
    start_in(1, 1)

    def pair(jj, carry):
        for b in (0, 1):
            j = jj * 2 + b
            wait_in(b)

            @pl.when(j >= 2)
            def _():
                wait_out(b)

            process(b)
            start_out(j, b)

            @pl.when(j + 2 < nchunks)
            def _():
                start_in(j + 2, b)
        return carry

    lax.fori_loop(0, nchunks // 2, pair, 0)
    wait_out(0)
    wait_out(1)


_sc_gather = pl.kernel(
    _gather_body,
    mesh=_MESH,
    out_type=jax.ShapeDtypeStruct((E,), jnp.float32),
    scratch_types=[
        pltpu.VMEM((NP,), jnp.float32),
        pltpu.VMEM((KG,), jnp.int32),
        pltpu.VMEM((KG,), jnp.int32),
        pltpu.VMEM((KG,), jnp.float32),
        pltpu.VMEM((KG,), jnp.float32),
        pltpu.SemaphoreType.DMA,
        pltpu.SemaphoreType.DMA,
        pltpu.SemaphoreType.DMA,
        pltpu.SemaphoreType.DMA,
    ],
    compiler_params=_SC_PARAMS,
)


def _scatter_body(dst_hbm, *rest):
    msg_refs = rest[:32]
    out_hbm, aggr_v, idx_v, val_v, sem0, sem1 = rest[32:]
    ch = _wid()

    zeros = jnp.zeros((16,), jnp.float32)

    def zinit(i, carry):
        aggr_v[pl.ds(i * 16, 16)] = zeros
        return carry

    lax.fori_loop(0, NP // 16, zinit, 0)

    nchunks = E // KS
    sems = (sem0, sem1)

    def start(j, b):
        pltpu.make_async_copy(
            dst_hbm.at[pl.ds(j * KS, KS)], idx_v.at[b], sems[b]).start()
        for cc in range(32):
            @pl.when(ch == cc)
            def _(_cc=cc):
                pltpu.make_async_copy(
                    msg_refs[_cc].at[pl.ds(j * KS, KS)], val_v.at[b],
                    sems[b]).start()

    def wait(b):
        pltpu.make_async_copy(
            dst_hbm.at[pl.ds(0, KS)], idx_v.at[b], sems[b]).wait()
        pltpu.make_async_copy(
            msg_refs[0].at[pl.ds(0, KS)], val_v.at[b], sems[b]).wait()

    def process(b):
        def group(g, acc):
            base = g * (16 * U)
            ivs = [idx_v[b, pl.ds(base + u * 16, 16)] for u in range(U)]
            mvs = [val_v[b, pl.ds(base + u * 16, 16)] for u in range(U)]
            curs = [plsc.load_gather(aggr_v, [ivs[u]]) for u in range(U)]
            news = [jnp.maximum(curs[u], mvs[u]) for u in range(U)]
            pends = [mvs[u] > curs[u] for u in range(U)]
            for u in range(U):
                plsc.store_scatter(aggr_v, [ivs[u]], news[u], mask=pends[u])
            chks = [plsc.load_gather(aggr_v, [ivs[u]]) for u in range(U)]
            for u in range(U):
                acc = jnp.logical_or(
                    acc, jnp.logical_and(pends[u], chks[u] < news[u]))
            return acc

        acc = lax.fori_loop(0, KS // (16 * U), group,
                            jnp.zeros((16,), jnp.bool_))

        def walk_cond(a):
            return jnp.any(a)

        def walk(a):
            def wgroup(g, acc2):
                base = g * (16 * U)
                for u in range(U):
                    iv = idx_v[b, pl.ds(base + u * 16, 16)]
                    mv = val_v[b, pl.ds(base + u * 16, 16)]
                    cell = plsc.load_gather(aggr_v, [iv])
                    pend = mv > cell
                    plsc.store_scatter(aggr_v, [iv], mv, mask=pend)
                    chk = plsc.load_gather(aggr_v, [iv])
                    acc2 = jnp.logical_or(
                        acc2, jnp.logical_and(pend, chk < mv))
                return acc2

            return lax.fori_loop(0, KS // (16 * U), wgroup,
                                 jnp.zeros((16,), jnp.bool_))

        lax.while_loop(walk_cond, walk, acc)

    start(0, 0)
    start(1, 1)

    def pair(jj, carry):
        for b in (0, 1):
            j = jj * 2 + b
            wait(b)
            process(b)

            @pl.when(j + 2 < nchunks)
            def _():
                start(j + 2, b)
        return carry

    lax.fori_loop(0, nchunks // 2, pair, 0)
    pltpu.sync_copy(aggr_v, out_hbm.at[pl.ds(ch * NP, NP)])


_sc_scatter_max = pl.kernel(
    _scatter_body,
    mesh=_MESH,
    out_type=jax.ShapeDtypeStruct((32 * NP,), jnp.float32),
    scratch_types=[
        pltpu.VMEM((NP,), jnp.float32),
        pltpu.VMEM((2, KS), jnp.int32),
        pltpu.VMEM((2, KS), jnp.float32),
        pltpu.SemaphoreType.DMA,
        pltpu.SemaphoreType.DMA,
    ],
    compiler_params=_SC_PARAMS,
)


def _edge_mlp_body(g0, g1, t, ea, w1a_r, b1a_r, w1b_r, b1b_r, *outs):
    x3 = jnp.concatenate(
        [g0[...].reshape(1, EB), g1[...].reshape(1, EB), t[...].reshape(1, EB)],
        axis=0,
    )
    w1a = w1a_r[...]
    h = jax.lax.dot_general(w1a[:3, :], x3, (((0,), (0,)), ((), ())),
                            preferred_element_type=jnp.float32)
    h = h + jax.lax.dot_general(w1a[3:, :], ea[...], (((0,), (0,)), ((), ())),
                                preferred_element_type=jnp.float32)
    h = jnp.maximum(h + b1a_r[...].reshape(16, 1), 0.0)
    m = jnp.maximum(
        jax.lax.dot_general(w1b_r[...], h, (((0,), (0,)), ((), ())),
                            preferred_element_type=jnp.float32)
        + b1b_r[...].reshape(32, 1), 0.0)
    for c in range(32):
        outs[c][...] = m[c, :]


def _edge_mlp(g0, g1, t, ea_t, w1a, b1a, w1b, b1b):
    grid = (E // EB,)
    vec = pl.BlockSpec((EB,), lambda i: (i,))
    return pl.pallas_call(
        _edge_mlp_body,
        grid=grid,
        in_specs=[
            vec, vec, vec,
            pl.BlockSpec((2, EB), lambda i: (0, i)),
            pl.BlockSpec((5, 16), lambda i: (0, 0)),
            pl.BlockSpec((16,), lambda i: (0,)),
            pl.BlockSpec((16, 32), lambda i: (0, 0)),
            pl.BlockSpec((32,), lambda i: (0,)),
        ],
        out_specs=[pl.BlockSpec((EB,), lambda i: (i,)) for _ in range(32)],
        out_shape=[jax.ShapeDtypeStruct((E,), jnp.float32) for _ in range(32)],
    )(g0, g1, t, ea_t, w1a, b1a, w1b, b1b)


def _node_mlp_body(x_ref, aggr_ref, w2a_ref, b2a_ref, w2b_ref, b2b_ref, out_ref):
    xa = x_ref[...]
    ag = aggr_ref[...]
    wa = w2a_ref[...]
    h = jax.lax.dot_general(wa[:3, :], xa, (((0,), (0,)), ((), ())),
                            preferred_element_type=jnp.float32)
    h = h + jax.lax.dot_general(wa[3:, :], ag, (((0,), (0,)), ((), ())),
                                preferred_element_type=jnp.float32)
    h = jnp.maximum(h + b2a_ref[...].reshape(16, 1), 0.0)
    o = jax.lax.dot_general(w2b_ref[...], h, (((0,), (0,)), ((), ())),
                            preferred_element_type=jnp.float32)
    out_ref[...] = jax.nn.sigmoid(o + b2b_ref[...].reshape(1, 1))


def _node_mlp(x_t, aggr_t, w2a, b2a, w2b, b2b):
    NB = 12544
    grid = (NP // NB,)
    return pl.pallas_call(
        _node_mlp_body,
        grid=grid,
        in_specs=[
            pl.BlockSpec((3, NB), lambda i: (0, i)),
            pl.BlockSpec((32, NB), lambda i: (0, i)),
            pl.BlockSpec((35, 16), lambda i: (0, 0)),
            pl.BlockSpec((16,), lambda i: (0,)),
            pl.BlockSpec((16, 1), lambda i: (0, 0)),
            pl.BlockSpec((1,), lambda i: (0,)),
        ],
        out_specs=pl.BlockSpec((1, NB), lambda i: (0, i)),
        out_shape=jax.ShapeDtypeStruct((1, NP), jnp.float32),
    )(x_t, aggr_t, w2a, b2a, w2b, b2b)


def kernel(x, edge_index, edge_attr, w1a, b1a, w1b, b1b, w2a, b2a, w2b, b2b):
    src = edge_index[0].astype(jnp.int32)
    dst = edge_index[1].astype(jnp.int32)
    ea_t = edge_attr.T

    pad = (0, NP - N)
    g0 = _sc_gather(jnp.pad(x[:, 0], pad), src)
    g1 = _sc_gather(jnp.pad(x[:, 1], pad), src)
    col2_tab = jnp.pad(x[:, 2], pad)

    x_t = jnp.pad(x.T, ((0, 0), pad))

    for _ in range(3):
        t_e = _sc_gather(col2_tab, src)
        msgs = _edge_mlp(g0, g1, t_e, ea_t, w1a, b1a, w1b, b1b)
        aggr_flat = _sc_scatter_max(dst, *msgs)
        aggr_t = aggr_flat.reshape(32, NP)
        comb = _node_mlp(x_t, aggr_t, w2a, b2a, w2b, b2b)
        col2_tab = comb.reshape(NP)
        x_t = x_t.at[2, :].set(comb[0])

    out = jnp.concatenate([x[:, :2], col2_tab[:N, None]], axis=1)
    return out
